# baseline (speedup 1.0000x reference)
import jax
import jax.numpy as jnp
from jax.experimental import pallas as pl
from jax.experimental.pallas import tpu as pltpu

_NEW = 9
_NBUF = 4


def _build_mt(H_z, H_x, qubit_rows, qubit_cols, qubit_src_idx,
              z_stab_rows, z_stab_cols, z_stab_src_idx,
              x_stab_rows, x_stab_cols, x_stab_src_idx, dtype):
    n_z = H_z.shape[0]
    nsyn = n_z + H_x.shape[0]
    MT = jnp.zeros((_NEW, _NEW, 6, nsyn + 1), dtype=dtype)
    MT = MT.at[z_stab_rows, z_stab_cols, 2, z_stab_src_idx].set(2.0)
    MT = MT.at[z_stab_rows, z_stab_cols, 2, nsyn].set(-1.0)
    MT = MT.at[x_stab_rows, x_stab_cols, 3, n_z + x_stab_src_idx].set(2.0)
    MT = MT.at[x_stab_rows, x_stab_cols, 3, nsyn].set(-1.0)
    MT = MT.at[qubit_rows, qubit_cols, 4, :n_z].set(
        H_z[:, qubit_src_idx].T.astype(dtype) / 4.0)
    MT = MT.at[qubit_rows, qubit_cols, 5, n_z:nsyn].set(
        H_x[:, qubit_src_idx].T.astype(dtype) / 4.0)
    return MT


def _body(nsteps, blk, s_ref, mt_ref, o_hbm, scratch, sems):
    i = pl.program_id(0)
    slot = jax.lax.rem(i, _NBUF)

    @pl.when(i >= _NBUF)
    def _():
        pltpu.make_async_copy(
            scratch.at[slot],
            o_hbm.at[:, :, :, pl.ds((i - _NBUF) * blk, blk)],
            sems.at[slot],
        ).wait()

    s = s_ref[...]
    for r in range(_NEW):
        for c in range(_NEW):
            scratch[slot, r, c] = jax.lax.dot_general(
                mt_ref[r, c], s, (((1,), (0,)), ((), ())),
                preferred_element_type=jnp.float32)

    pltpu.make_async_copy(
        scratch.at[slot],
        o_hbm.at[:, :, :, pl.ds(i * blk, blk)],
        sems.at[slot],
    ).start()

    @pl.when(i == nsteps - 1)
    def _():
        for k in range(_NBUF):
            step = nsteps - _NBUF + k
            pltpu.make_async_copy(
                scratch.at[k],
                o_hbm.at[:, :, :, pl.ds(step * blk, blk)],
                sems.at[k],
            ).wait()


def kernel(syndrome, H_z, H_x, qubit_rows, qubit_cols, qubit_src_idx,
           z_stab_rows, z_stab_cols, z_stab_src_idx,
           x_stab_rows, x_stab_cols, x_stab_src_idx):
    B = syndrome.shape[0]
    nsyn = H_z.shape[0] + H_x.shape[0]
    MT = _build_mt(H_z, H_x, qubit_rows, qubit_cols, qubit_src_idx,
                   z_stab_rows, z_stab_cols, z_stab_src_idx,
                   x_stab_rows, x_stab_cols, x_stab_src_idx,
                   syndrome.dtype)
    sA = jnp.concatenate(
        [syndrome.T, jnp.ones((1, B), dtype=syndrome.dtype)], axis=0)
    BLK = 1024
    nsteps = B // BLK
    import functools
    outT = pl.pallas_call(
        functools.partial(_body, nsteps, BLK),
        grid=(nsteps,),
        in_specs=[
            pl.BlockSpec((nsyn + 1, BLK), lambda i: (0, i)),
            pl.BlockSpec((_NEW, _NEW, 6, nsyn + 1), lambda i: (0, 0, 0, 0)),
        ],
        out_specs=pl.BlockSpec(memory_space=pltpu.MemorySpace.HBM),
        out_shape=jax.ShapeDtypeStruct((_NEW, _NEW, 6, B), syndrome.dtype),
        scratch_shapes=[
            pltpu.VMEM((_NBUF, _NEW, _NEW, 6, BLK), syndrome.dtype),
            pltpu.SemaphoreType.DMA((_NBUF,)),
        ],
    )(sA, MT)
    return jnp.transpose(outT, (3, 2, 0, 1))


# speedup vs baseline: 17.6454x; 17.6454x over previous
import jax
import jax.numpy as jnp
from jax.experimental import pallas as pl
from jax.experimental.pallas import tpu as pltpu

_NEW = 9


def _body(s_ref, hz_ref, hx_ref, qr, qc, qs, zr, zc, zs, xr, xc, xs,
          o_ref, mt):
    i = pl.program_id(0)

    @pl.when(i == 0)
    def _build():
        mt[...] = jnp.zeros(mt.shape, dtype=mt.dtype)
        lane = jax.lax.iota(jnp.int32, 17)
        bias_row = jnp.where(lane == 16, -1.0, 0.0).astype(mt.dtype)
        for k in range(zr.shape[0]):
            row = jnp.where(lane == zs[k], 2.0, 0.0).astype(mt.dtype)
            mt[zr[k], zc[k], 2] = row + bias_row
        for k in range(xr.shape[0]):
            row = jnp.where(lane == 8 + xs[k], 2.0, 0.0).astype(mt.dtype)
            mt[xr[k], xc[k], 3] = row + bias_row
        for k in range(qr.shape[0]):
            row4 = jnp.zeros((17,), dtype=mt.dtype)
            row5 = jnp.zeros((17,), dtype=mt.dtype)
            for j in range(hz_ref.shape[0]):
                row4 = jnp.where(lane == j, hz_ref[j, qs[k]] * 0.25, row4)
                row5 = jnp.where(lane == 8 + j, hx_ref[j, qs[k]] * 0.25, row5)
            mt[qr[k], qc[k], 4] = row4
            mt[qr[k], qc[k], 5] = row5

    s = s_ref[...]
    for c in range(_NEW):
        o_ref[0, c] = jax.lax.dot_general(
            mt[i, c], s, (((1,), (0,)), ((), ())),
            preferred_element_type=jnp.float32)


def kernel(syndrome, H_z, H_x, qubit_rows, qubit_cols, qubit_src_idx,
           z_stab_rows, z_stab_cols, z_stab_src_idx,
           x_stab_rows, x_stab_cols, x_stab_src_idx):
    B = syndrome.shape[0]
    nsyn = H_z.shape[0] + H_x.shape[0]
    sA = jnp.concatenate(
        [syndrome.T, jnp.ones((1, B), dtype=syndrome.dtype)], axis=0)
    smem = pl.BlockSpec(memory_space=pltpu.MemorySpace.SMEM)
    outT = pl.pallas_call(
        _body,
        grid=(_NEW,),
        in_specs=[pl.BlockSpec((nsyn + 1, B), lambda i: (0, 0))] + [smem] * 11,
        out_specs=pl.BlockSpec((1, _NEW, 6, B), lambda i: (i, 0, 0, 0)),
        out_shape=jax.ShapeDtypeStruct((_NEW, _NEW, 6, B), syndrome.dtype),
        scratch_shapes=[pltpu.VMEM((_NEW, _NEW, 6, nsyn + 1), syndrome.dtype)],
    )(sA, H_z, H_x, qubit_rows, qubit_cols, qubit_src_idx,
      z_stab_rows, z_stab_cols, z_stab_src_idx,
      x_stab_rows, x_stab_cols, x_stab_src_idx)
    return jnp.transpose(outT, (3, 2, 0, 1))
